# Initial kernel scaffold; baseline (speedup 1.0000x reference)
import functools

import jax
import jax.numpy as jnp
from jax import lax
from jax.experimental import pallas as pl
from jax.experimental.pallas import tpu as pltpu
from jax.experimental.pallas import tpu_sc as plsc

N = 10000
H = 128
OUT = 40
NC = 2
NS = 16
NW = NC * NS
CH = 128
N_PAD = 10240
ROWS = N_PAD // NS
DEG_W = 16
ZB = 64


def _sc_degree(dst2d, cpw):
    mesh = plsc.VectorSubcoreMesh(core_axis_name="c", subcore_axis_name="s")

    @functools.partial(
        pl.kernel,
        out_type=jax.ShapeDtypeStruct((NC * N_PAD, DEG_W), jnp.float32),
        mesh=mesh,
        scratch_types=[
            pltpu.VMEM((1, CH), jnp.int32),
            pltpu.VMEM((CH, DEG_W), jnp.float32),
            pltpu.VMEM((ZB, DEG_W), jnp.float32),
            pltpu.VMEM_SHARED((N_PAD, DEG_W), jnp.float32),
            pltpu.SemaphoreType.DMA,
        ],
    )
    def k(dst_hbm, out_hbm, didx, ones_v, zero_v, acc, sem):
        c = lax.axis_index("c")
        s = lax.axis_index("s")
        wid = s * NC + c

        @pl.loop(0, CH)
        def _(i):
            ones_v[i, :] = jnp.ones((DEG_W,), jnp.float32)

        @pl.loop(0, ZB)
        def _(i):
            zero_v[i, :] = jnp.zeros((DEG_W,), jnp.float32)

        base = s * ROWS

        @pl.loop(0, ROWS, step=ZB)
        def _(r):
            pltpu.sync_copy(zero_v, acc.at[pl.ds(base + r, ZB)])

        plsc.subcore_barrier()

        @pl.loop(0, cpw)
        def _(j):
            pltpu.sync_copy(dst_hbm.at[pl.ds(wid * cpw + j, 1)], didx)
            pltpu.sync_copy(ones_v, acc.at[didx.at[0]], add=True)

        plsc.subcore_barrier()
        pltpu.sync_copy(acc.at[pl.ds(base, ROWS)],
                        out_hbm.at[pl.ds(c * N_PAD + base, ROWS)])

    return k(dst2d)


def _sc_agg(h, src2d, dst2d, cpw):
    mesh = plsc.VectorSubcoreMesh(core_axis_name="c", subcore_axis_name="s")

    @functools.partial(
        pl.kernel,
        out_type=jax.ShapeDtypeStruct((NC * N_PAD, H), jnp.float32),
        mesh=mesh,
        scratch_types=[
            pltpu.VMEM((1, CH), jnp.int32),
            pltpu.VMEM((1, CH), jnp.int32),
            pltpu.VMEM((CH, H), jnp.float32),
            pltpu.VMEM((ZB, H), jnp.float32),
            pltpu.VMEM_SHARED((N_PAD, H), jnp.float32),
            pltpu.SemaphoreType.DMA,
        ],
    )
    def k(h_hbm, src_hbm, dst_hbm, out_hbm, sidx, didx, rows_v, zero_v, acc, sem):
        c = lax.axis_index("c")
        s = lax.axis_index("s")
        wid = s * NC + c

        @pl.loop(0, ZB)
        def _(i):
            @pl.loop(0, H, step=16)
            def _(j):
                zero_v[i, pl.ds(j, 16)] = jnp.zeros((16,), jnp.float32)

        base = s * ROWS

        @pl.loop(0, ROWS, step=ZB)
        def _(r):
            pltpu.sync_copy(zero_v, acc.at[pl.ds(base + r, ZB)])

        plsc.subcore_barrier()

        @pl.loop(0, cpw)
        def _(j):
            pltpu.sync_copy(src_hbm.at[pl.ds(wid * cpw + j, 1)], sidx)
            pltpu.sync_copy(dst_hbm.at[pl.ds(wid * cpw + j, 1)], didx)
            pltpu.async_copy(h_hbm.at[sidx.at[0]], rows_v, sem).wait()
            pltpu.sync_copy(rows_v, acc.at[didx.at[0]], add=True)

        plsc.subcore_barrier()
        pltpu.sync_copy(acc.at[pl.ds(base, ROWS)],
                        out_hbm.at[pl.ds(c * N_PAD + base, ROWS)])

    return k(h, src2d, dst2d)


BR = 400


def _tc_layer_body(a0, a1, d0, d1, h_ref, wl, blr, wr, o_ref):
    cnt = d0[:, 0:1] + d1[:, 0:1]
    inv = 1.0 / jnp.maximum(cnt, 1.0)
    mean = (a0[...] + a1[...]) * inv
    acc = lax.dot_general(mean, wl[...], (((1,), (1,)), ((), ())),
                          preferred_element_type=jnp.float32,
                          precision=lax.Precision.HIGHEST)
    acc = acc + blr[...]
    acc = acc + lax.dot_general(h_ref[...], wr[...], (((1,), (1,)), ((), ())),
                                preferred_element_type=jnp.float32,
                                precision=lax.Precision.HIGHEST)
    o_ref[...] = jnp.maximum(acc, 0.0)


def _tc_layer(a0, a1, d0, d1, h, Wl, bl, Wr):
    nb = N // BR
    return pl.pallas_call(
        _tc_layer_body,
        grid=(nb,),
        in_specs=[
            pl.BlockSpec((BR, H), lambda i: (i, 0)),
            pl.BlockSpec((BR, H), lambda i: (i, 0)),
            pl.BlockSpec((BR, DEG_W), lambda i: (i, 0)),
            pl.BlockSpec((BR, DEG_W), lambda i: (i, 0)),
            pl.BlockSpec((BR, H), lambda i: (i, 0)),
            pl.BlockSpec((H, H), lambda i: (0, 0)),
            pl.BlockSpec((1, H), lambda i: (0, 0)),
            pl.BlockSpec((H, H), lambda i: (0, 0)),
        ],
        out_specs=pl.BlockSpec((BR, H), lambda i: (i, 0)),
        out_shape=jax.ShapeDtypeStruct((N, H), jnp.float32),
    )(a0, a1, d0, d1, h, Wl, bl, Wr)


def _tc_final_body(h1, h2, h3, w1, w2, w3, br, o_ref):
    acc = lax.dot_general(h1[...], w1[...], (((1,), (1,)), ((), ())),
                          preferred_element_type=jnp.float32,
                          precision=lax.Precision.HIGHEST)
    acc = acc + lax.dot_general(h2[...], w2[...], (((1,), (1,)), ((), ())),
                                preferred_element_type=jnp.float32,
                                precision=lax.Precision.HIGHEST)
    acc = acc + lax.dot_general(h3[...], w3[...], (((1,), (1,)), ((), ())),
                                preferred_element_type=jnp.float32,
                                precision=lax.Precision.HIGHEST)
    o_ref[...] = acc + br[...]


def _tc_final(h1, h2, h3, w1, w2, w3, fc_b):
    nb = N // BR
    return pl.pallas_call(
        _tc_final_body,
        grid=(nb,),
        in_specs=[
            pl.BlockSpec((BR, H), lambda i: (i, 0)),
            pl.BlockSpec((BR, H), lambda i: (i, 0)),
            pl.BlockSpec((BR, H), lambda i: (i, 0)),
            pl.BlockSpec((OUT, H), lambda i: (0, 0)),
            pl.BlockSpec((OUT, H), lambda i: (0, 0)),
            pl.BlockSpec((OUT, H), lambda i: (0, 0)),
            pl.BlockSpec((1, OUT), lambda i: (0, 0)),
        ],
        out_specs=pl.BlockSpec((BR, OUT), lambda i: (i, 0)),
        out_shape=jax.ShapeDtypeStruct((N, OUT), jnp.float32),
    )(h1, h2, h3, w1, w2, w3, fc_b)


def kernel(x, edge_index, Wl0, bl0, Wr0, Wl1, bl1, Wr1, Wl2, bl2, Wr2, fc_W, fc_b):
    src = edge_index[0]
    dst = edge_index[1]
    e = src.shape[0]
    cpw = -(-e // (NW * CH))
    e_pad = NW * CH * cpw
    src_p = jnp.concatenate(
        [src, jnp.zeros((e_pad - e,), jnp.int32)]).reshape(-1, CH)
    dst_p = jnp.concatenate(
        [dst, jnp.full((e_pad - e,), N, jnp.int32)]).reshape(-1, CH)

    degs = _sc_degree(dst_p, cpw)
    d0 = degs[0:N]
    d1 = degs[N_PAD:N_PAD + N]

    h = x
    hs = []
    for (Wl, bl, Wr) in ((Wl0, bl0, Wr0), (Wl1, bl1, Wr1), (Wl2, bl2, Wr2)):
        parts = _sc_agg(h, src_p, dst_p, cpw)
        h = _tc_layer(parts[0:N], parts[N_PAD:N_PAD + N], d0, d1, h,
                      Wl, bl.reshape(1, H), Wr)
        hs.append(h)

    return _tc_final(hs[0], hs[1], hs[2],
                     fc_W[:, 0:H], fc_W[:, H:2 * H], fc_W[:, 2 * H:3 * H],
                     fc_b.reshape(1, OUT))


# speedup vs baseline: 3.2078x; 3.2078x over previous
import functools

import jax
import jax.numpy as jnp
from jax import lax
from jax.experimental import pallas as pl
from jax.experimental.pallas import tpu as pltpu
from jax.experimental.pallas import tpu_sc as plsc

N = 10000
H = 128
OUT = 40
NC = 2
NS = 16
NW = NC * NS
CH = 128
N_PAD = 10240
ROWS = N_PAD // NS
DEG_W = 16
ZB = 64


def _sc_agg_build(cpw, gather):
    mesh = plsc.VectorSubcoreMesh(core_axis_name="c", subcore_axis_name="s")
    scratch = [
        pltpu.VMEM((1, CH), jnp.int32),
        pltpu.VMEM((CH, H), jnp.float32),
        pltpu.VMEM((ZB, H), jnp.float32),
        pltpu.VMEM_SHARED((N_PAD, H), jnp.float32),
        pltpu.SemaphoreType.DMA,
    ]
    if gather:
        scratch.insert(0, pltpu.VMEM((1, CH), jnp.int32))

    def body(refs):
        if gather:
            h_hbm, src_hbm, dst_hbm, out_hbm, sidx, didx, rows_v, zero_v, acc, sem = refs
        else:
            dst_hbm, out_hbm, didx, rows_v, zero_v, acc, sem = refs
        c = lax.axis_index("c")
        s = lax.axis_index("s")
        wid = s * NC + c

        @pl.loop(0, ZB)
        def _(i):
            @pl.loop(0, H, step=16)
            def _(j):
                zero_v[i, pl.ds(j, 16)] = jnp.zeros((16,), jnp.float32)

        if not gather:
            @pl.loop(0, CH)
            def _(i):
                @pl.loop(0, H, step=16)
                def _(j):
                    rows_v[i, pl.ds(j, 16)] = jnp.ones((16,), jnp.float32)

        base = s * ROWS

        @pl.loop(0, ROWS, step=ZB)
        def _(r):
            pltpu.sync_copy(zero_v, acc.at[pl.ds(base + r, ZB)])

        plsc.subcore_barrier()

        @pl.loop(0, cpw)
        def _(j):
            pltpu.sync_copy(dst_hbm.at[pl.ds(wid * cpw + j, 1)], didx)
            if gather:
                pltpu.sync_copy(src_hbm.at[pl.ds(wid * cpw + j, 1)], sidx)
                pltpu.async_copy(h_hbm.at[sidx.at[0]], rows_v, sem).wait()
            pltpu.sync_copy(rows_v, acc.at[didx.at[0]], add=True)

        plsc.subcore_barrier()
        pltpu.sync_copy(acc.at[pl.ds(base, ROWS)],
                        out_hbm.at[pl.ds(c * N_PAD + base, ROWS)])

    out_type = jax.ShapeDtypeStruct((NC * N_PAD, H), jnp.float32)

    @functools.partial(pl.kernel, out_type=out_type, mesh=mesh,
                       scratch_types=scratch)
    def k(*refs):
        body(refs)

    return k


def _sc_degree(dst2d, cpw):
    return _sc_agg_build(cpw, gather=False)(dst2d)


def _sc_agg(h, src2d, dst2d, cpw):
    return _sc_agg_build(cpw, gather=True)(h, src2d, dst2d)


BR = 400


def _tc_layer_body(a0, a1, d0, d1, h_ref, wl, blr, wr, o_ref):
    cnt = d0[:, 0:1] + d1[:, 0:1]
    inv = 1.0 / jnp.maximum(cnt, 1.0)
    mean = (a0[...] + a1[...]) * inv
    acc = lax.dot_general(mean, wl[...], (((1,), (1,)), ((), ())),
                          preferred_element_type=jnp.float32,
                          precision=lax.Precision.HIGHEST)
    acc = acc + blr[...]
    acc = acc + lax.dot_general(h_ref[...], wr[...], (((1,), (1,)), ((), ())),
                                preferred_element_type=jnp.float32,
                                precision=lax.Precision.HIGHEST)
    o_ref[...] = jnp.maximum(acc, 0.0)


def _tc_layer(a0, a1, d0, d1, h, Wl, bl, Wr):
    nb = N // BR
    return pl.pallas_call(
        _tc_layer_body,
        grid=(nb,),
        in_specs=[
            pl.BlockSpec((BR, H), lambda i: (i, 0)),
            pl.BlockSpec((BR, H), lambda i: (i, 0)),
            pl.BlockSpec((BR, H), lambda i: (i, 0)),
            pl.BlockSpec((BR, H), lambda i: (i, 0)),
            pl.BlockSpec((BR, H), lambda i: (i, 0)),
            pl.BlockSpec((H, H), lambda i: (0, 0)),
            pl.BlockSpec((1, H), lambda i: (0, 0)),
            pl.BlockSpec((H, H), lambda i: (0, 0)),
        ],
        out_specs=pl.BlockSpec((BR, H), lambda i: (i, 0)),
        out_shape=jax.ShapeDtypeStruct((N, H), jnp.float32),
    )(a0, a1, d0, d1, h, Wl, bl, Wr)


def _tc_final_body(h1, h2, h3, w1, w2, w3, br, o_ref):
    acc = lax.dot_general(h1[...], w1[...], (((1,), (1,)), ((), ())),
                          preferred_element_type=jnp.float32,
                          precision=lax.Precision.HIGHEST)
    acc = acc + lax.dot_general(h2[...], w2[...], (((1,), (1,)), ((), ())),
                                preferred_element_type=jnp.float32,
                                precision=lax.Precision.HIGHEST)
    acc = acc + lax.dot_general(h3[...], w3[...], (((1,), (1,)), ((), ())),
                                preferred_element_type=jnp.float32,
                                precision=lax.Precision.HIGHEST)
    o_ref[...] = acc + br[...]


def _tc_final(h1, h2, h3, w1, w2, w3, fc_b):
    nb = N // BR
    return pl.pallas_call(
        _tc_final_body,
        grid=(nb,),
        in_specs=[
            pl.BlockSpec((BR, H), lambda i: (i, 0)),
            pl.BlockSpec((BR, H), lambda i: (i, 0)),
            pl.BlockSpec((BR, H), lambda i: (i, 0)),
            pl.BlockSpec((OUT, H), lambda i: (0, 0)),
            pl.BlockSpec((OUT, H), lambda i: (0, 0)),
            pl.BlockSpec((OUT, H), lambda i: (0, 0)),
            pl.BlockSpec((1, OUT), lambda i: (0, 0)),
        ],
        out_specs=pl.BlockSpec((BR, OUT), lambda i: (i, 0)),
        out_shape=jax.ShapeDtypeStruct((N, OUT), jnp.float32),
    )(h1, h2, h3, w1, w2, w3, fc_b)


def kernel(x, edge_index, Wl0, bl0, Wr0, Wl1, bl1, Wr1, Wl2, bl2, Wr2, fc_W, fc_b):
    src = edge_index[0]
    dst = edge_index[1]
    e = src.shape[0]
    cpw = -(-e // (NW * CH))
    e_pad = NW * CH * cpw
    src_p = jnp.concatenate(
        [src, jnp.zeros((e_pad - e,), jnp.int32)]).reshape(-1, CH)
    dst_p = jnp.concatenate(
        [dst, jnp.full((e_pad - e,), N, jnp.int32)]).reshape(-1, CH)

    degs = _sc_degree(dst_p, cpw)
    d0 = degs[0:N]
    d1 = degs[N_PAD:N_PAD + N]

    h = x
    hs = []
    for (Wl, bl, Wr) in ((Wl0, bl0, Wr0), (Wl1, bl1, Wr1), (Wl2, bl2, Wr2)):
        parts = _sc_agg(h, src_p, dst_p, cpw)
        h = _tc_layer(parts[0:N], parts[N_PAD:N_PAD + N], d0, d1, h,
                      Wl, bl.reshape(1, H), Wr)
        hs.append(h)

    return _tc_final(hs[0], hs[1], hs[2],
                     fc_W[:, 0:H], fc_W[:, H:2 * H], fc_W[:, 2 * H:3 * H],
                     fc_b.reshape(1, OUT))
